# Initial kernel scaffold; baseline (speedup 1.0000x reference)
import jax
import jax.numpy as jnp
from jax.experimental import pallas as pl


def kernel(points, features, W0, W1, W2, Wsc, g0, b0, g1, b1, g2, b2, gsc, bsc):
    raise NotImplementedError("write your pallas kernel here")


# speedup vs baseline: 1.9896x; 1.9896x over previous
import functools

import jax
import jax.numpy as jnp
from jax import lax
from jax.experimental import pallas as pl
from jax.experimental.pallas import tpu as pltpu
from jax.experimental.pallas import tpu_sc as plsc

B = 32
N = 1024
C = 64
K = 16
EPS = 1e-5
M = B * N * K
BN = B * N
CNT_EDGE = float(M)
CNT_PT = float(BN)

_PREC = lax.Precision.HIGHEST


def _knn_body(points_ref, out_ref):
    b = pl.program_id(0)
    x = points_ref[0]
    xx = jnp.sum(x * x, axis=0, keepdims=True)
    xtx = lax.dot_general(x, x, (((0,), (0,)), ((), ())),
                          preferred_element_type=jnp.float32)
    inner = -2.0 * xtx
    pd = (-xx - inner) - jnp.swapaxes(xx, 0, 1)
    iota = lax.broadcasted_iota(jnp.int32, (N, N), 1)
    base = b * N
    cols = []
    for t in range(K + 1):
        m = jnp.max(pd, axis=1, keepdims=True)
        sel = pd == m
        cand = jnp.min(jnp.where(sel, iota, jnp.int32(1 << 30)),
                       axis=1, keepdims=True)
        if t > 0:
            cols.append(cand + base)
        if t < K:
            pd = jnp.where(iota == cand, -jnp.inf, pd)
    out_ref[0] = jnp.concatenate(cols, axis=1)


def _knn(points):
    return pl.pallas_call(
        _knn_body,
        grid=(B,),
        in_specs=[pl.BlockSpec((1, 3, N), lambda b: (b, 0, 0))],
        out_specs=pl.BlockSpec((1, N, K), lambda b: (b, 0, 0)),
        out_shape=jax.ShapeDtypeStruct((B, N, K), jnp.int32),
    )(points)


def _prep_body(f_ref, a_ref, bm_ref, wsc_ref, u_ref, v_ref, sc_ref, scst_ref):
    b = pl.program_id(0)
    f = f_ref[0]

    def proj(w):
        return lax.dot_general(f, w, (((0,), (1,)), ((), ())),
                               preferred_element_type=jnp.float32,
                               precision=_PREC)

    u_ref[0] = proj(a_ref[...])
    v_ref[0] = proj(bm_ref[...])
    sc = proj(wsc_ref[...])
    sc_ref[0] = sc

    @pl.when(b == 0)
    def _():
        scst_ref[...] = jnp.zeros_like(scst_ref)

    s = jnp.sum(sc, axis=0, keepdims=True)
    ss = jnp.sum(sc * sc, axis=0, keepdims=True)
    scst_ref[...] += jnp.concatenate([s, ss], axis=0)


def _prep(features, A, Bm, Wsc):
    return pl.pallas_call(
        _prep_body,
        grid=(B,),
        in_specs=[
            pl.BlockSpec((1, C, N), lambda b: (b, 0, 0)),
            pl.BlockSpec((C, C), lambda b: (0, 0)),
            pl.BlockSpec((C, C), lambda b: (0, 0)),
            pl.BlockSpec((C, C), lambda b: (0, 0)),
        ],
        out_specs=[
            pl.BlockSpec((1, N, C), lambda b: (b, 0, 0)),
            pl.BlockSpec((1, N, C), lambda b: (b, 0, 0)),
            pl.BlockSpec((1, N, C), lambda b: (b, 0, 0)),
            pl.BlockSpec((2, C), lambda b: (0, 0)),
        ],
        out_shape=[
            jax.ShapeDtypeStruct((B, N, C), jnp.float32),
            jax.ShapeDtypeStruct((B, N, C), jnp.float32),
            jax.ShapeDtypeStruct((B, N, C), jnp.float32),
            jax.ShapeDtypeStruct((2, C), jnp.float32),
        ],
    )(features, A, Bm, Wsc)


_SC_CHUNK = 128


def _sc_gather(vt_flat, idxf):
    info = plsc.get_sparse_core_info()
    nc, ns = info.num_cores, info.num_subcores
    nw = nc * ns
    rows_per_w = M // nw
    iters = rows_per_w // _SC_CHUNK
    mesh = plsc.VectorSubcoreMesh(core_axis_name="c", subcore_axis_name="s")

    @functools.partial(
        pl.kernel, mesh=mesh,
        compiler_params=pltpu.CompilerParams(use_tc_tiling_on_sc=False),
        out_type=jax.ShapeDtypeStruct((M, C), jnp.float32),
        scratch_types=[
            pltpu.VMEM((_SC_CHUNK,), jnp.int32),
            pltpu.VMEM((_SC_CHUNK, C), jnp.float32),
            pltpu.SemaphoreType.DMA,
        ],
    )
    def k(vt_hbm, idx_hbm, out_hbm, idx_v, rows_v, sem):
        wid = lax.axis_index("s") * nc + lax.axis_index("c")
        base = wid * rows_per_w

        def body(i, carry):
            off = base + i * _SC_CHUNK
            pltpu.sync_copy(idx_hbm.at[pl.ds(off, _SC_CHUNK)], idx_v)
            pltpu.async_copy(vt_hbm.at[idx_v], rows_v, sem).wait()
            pltpu.sync_copy(rows_v, out_hbm.at[pl.ds(off, _SC_CHUNK)])
            return carry

        lax.fori_loop(0, iters, body, 0)

    return k(vt_flat, idxf)


def _bn_affine(st_ref, g, b, cnt):
    s = st_ref[0, :]
    ss = st_ref[1, :]
    mean = s / cnt
    var = ss / cnt - mean * mean
    inv = g * lax.rsqrt(var + EPS)
    return inv.reshape(1, C), (b - mean * inv).reshape(1, C)


_ROWS = 2048
_PTS = _ROWS // K
_STEPS = M // _ROWS


def _acc_stats(st_ref, y, first):
    @pl.when(first)
    def _():
        st_ref[...] = jnp.zeros_like(st_ref)

    s = jnp.sum(y, axis=0, keepdims=True)
    ss = jnp.sum(y * y, axis=0, keepdims=True)
    st_ref[...] += jnp.concatenate([s, ss], axis=0)


def _edge_in_specs(extra):
    return [
        pl.BlockSpec((_ROWS, C), lambda i: (i, 0)),
        pl.BlockSpec((_PTS, C), lambda i: (i, 0)),
    ] + extra


def _stat_spec():
    return pl.BlockSpec((2, C), lambda i: (0, 0))


def _w_spec():
    return pl.BlockSpec((C, C), lambda i: (0, 0))


def _p_spec():
    return pl.BlockSpec((8, C), lambda i: (0, 0))


def _y1(e_ref, u_ref):
    e3 = e_ref[...].reshape(_PTS, K, C)
    y1 = e3 + u_ref[...].reshape(_PTS, 1, C)
    return y1.reshape(_ROWS, C)


def _st1_body(e_ref, u_ref, st_ref):
    _acc_stats(st_ref, _y1(e_ref, u_ref), pl.program_id(0) == 0)


def _st1(E, Ut):
    return pl.pallas_call(
        _st1_body,
        grid=(_STEPS,),
        in_specs=_edge_in_specs([]),
        out_specs=_stat_spec(),
        out_shape=jax.ShapeDtypeStruct((2, C), jnp.float32),
    )(E, Ut)


def _mm(z, w_ref):
    return lax.dot_general(z, w_ref[...], (((1,), (1,)), ((), ())),
                           preferred_element_type=jnp.float32,
                           precision=_PREC)


def _st2_body(e_ref, u_ref, st1_ref, w1_ref, p_ref, st_ref):
    s0, t0 = _bn_affine(st1_ref, p_ref[0, :], p_ref[1, :], CNT_EDGE)
    z1 = jax.nn.relu(_y1(e_ref, u_ref) * s0 + t0)
    y2 = _mm(z1, w1_ref)
    _acc_stats(st_ref, y2, pl.program_id(0) == 0)


def _st2(E, Ut, st1, W1, P):
    return pl.pallas_call(
        _st2_body,
        grid=(_STEPS,),
        in_specs=_edge_in_specs([_stat_spec(), _w_spec(), _p_spec()]),
        out_specs=_stat_spec(),
        out_shape=jax.ShapeDtypeStruct((2, C), jnp.float32),
    )(E, Ut, st1, W1, P)


def _st3_body(e_ref, u_ref, st1_ref, st2_ref, w1_ref, w2_ref, p_ref, st_ref):
    s0, t0 = _bn_affine(st1_ref, p_ref[0, :], p_ref[1, :], CNT_EDGE)
    s1, t1 = _bn_affine(st2_ref, p_ref[2, :], p_ref[3, :], CNT_EDGE)
    z1 = jax.nn.relu(_y1(e_ref, u_ref) * s0 + t0)
    z2 = jax.nn.relu(_mm(z1, w1_ref) * s1 + t1)
    y3 = _mm(z2, w2_ref)
    _acc_stats(st_ref, y3, pl.program_id(0) == 0)


def _st3(E, Ut, st1, st2, W1, W2, P):
    return pl.pallas_call(
        _st3_body,
        grid=(_STEPS,),
        in_specs=_edge_in_specs([_stat_spec(), _stat_spec(), _w_spec(),
                                 _w_spec(), _p_spec()]),
        out_specs=_stat_spec(),
        out_shape=jax.ShapeDtypeStruct((2, C), jnp.float32),
    )(E, Ut, st1, st2, W1, W2, P)


def _fin_body(e_ref, u_ref, sct_ref, st1_ref, st2_ref, st3_ref, scst_ref,
              w1_ref, w2_ref, p_ref, out_ref):
    s0, t0 = _bn_affine(st1_ref, p_ref[0, :], p_ref[1, :], CNT_EDGE)
    s1, t1 = _bn_affine(st2_ref, p_ref[2, :], p_ref[3, :], CNT_EDGE)
    s2, t2 = _bn_affine(st3_ref, p_ref[4, :], p_ref[5, :], CNT_EDGE)
    ssc, tsc = _bn_affine(scst_ref, p_ref[6, :], p_ref[7, :], CNT_PT)
    z1 = jax.nn.relu(_y1(e_ref, u_ref) * s0 + t0)
    z2 = jax.nn.relu(_mm(z1, w1_ref) * s1 + t1)
    z3 = jax.nn.relu(_mm(z2, w2_ref) * s2 + t2)
    fts = jnp.mean(z3.reshape(_PTS, K, C), axis=1)
    sc = sct_ref[...] * ssc + tsc
    res = jax.nn.relu(sc + fts)
    out_ref[0] = jnp.swapaxes(res, 0, 1)


def _final(E, Ut, SCt, st1, st2, st3, scst, W1, W2, P):
    return pl.pallas_call(
        _fin_body,
        grid=(_STEPS,),
        in_specs=_edge_in_specs([
            pl.BlockSpec((_PTS, C), lambda i: (i, 0)),
            _stat_spec(), _stat_spec(), _stat_spec(), _stat_spec(),
            _w_spec(), _w_spec(), _p_spec(),
        ]),
        out_specs=pl.BlockSpec((1, C, _PTS), lambda i: (i // 8, 0, i % 8)),
        out_shape=jax.ShapeDtypeStruct((B, C, N), jnp.float32),
    )(E, Ut, SCt, st1, st2, st3, scst, W1, W2, P)


def kernel(points, features, W0, W1, W2, Wsc, g0, b0, g1, b1, g2, b2, gsc, bsc):
    A = W0[:, :C] - W0[:, C:]
    Bm = W0[:, C:]
    P = jnp.stack([g0, b0, g1, b1, g2, b2, gsc, bsc])

    idx = _knn(points)
    Ut, Vt, SCt, scst = _prep(features, A, Bm, Wsc)
    idxf = idx.reshape(-1)
    vt_flat = Vt.reshape(BN, C)
    ut_flat = Ut.reshape(BN, C)
    sct_flat = SCt.reshape(BN, C)

    E = _sc_gather(vt_flat, idxf)

    st1 = _st1(E, ut_flat)
    st2 = _st2(E, ut_flat, st1, W1, P)
    st3 = _st3(E, ut_flat, st1, st2, W1, W2, P)
    return _final(E, ut_flat, sct_flat, st1, st2, st3, scst, W1, W2, P)
